# baseline (device time: 31491 ns/iter reference)
import jax
import jax.numpy as jnp
from jax import lax
from jax.experimental import pallas as pl
from jax.experimental.pallas import tpu as pltpu

N_DEV = 8


def kernel(x, Wg, Wu, Wd):
    m, k = x.shape
    d = Wd.shape[1]

    def body(x_ref, wg_ref, wu_ref, wd_ref, out_ref, comm_ref, send_sems, recv_sems):
        my = lax.axis_index("i")
        left = lax.rem(my + N_DEV - 1, N_DEV)
        right = lax.rem(my + 1, N_DEV)

        barrier_sem = pltpu.get_barrier_semaphore()
        for nbr in (left, right):
            pl.semaphore_signal(
                barrier_sem, inc=1,
                device_id=(nbr,), device_id_type=pl.DeviceIdType.MESH,
            )
        pl.semaphore_wait(barrier_sem, 2)

        xb = x_ref[...].astype(jnp.bfloat16)
        gate = jnp.dot(xb, wg_ref[...].astype(jnp.bfloat16),
                       preferred_element_type=jnp.float32)
        up = jnp.dot(xb, wu_ref[...].astype(jnp.bfloat16),
                     preferred_element_type=jnp.float32)
        hidden = (gate * (up * jax.nn.sigmoid(up))).astype(jnp.bfloat16)
        partial = jnp.dot(hidden, wd_ref[...].astype(jnp.bfloat16),
                          preferred_element_type=jnp.float32)

        comm_ref[0, :, :] = partial.astype(jnp.bfloat16)
        acc = partial

        for h in range(N_DEV - 1):
            rdma = pltpu.make_async_remote_copy(
                src_ref=comm_ref.at[h],
                dst_ref=comm_ref.at[h + 1],
                send_sem=send_sems.at[h],
                recv_sem=recv_sems.at[h],
                device_id=(right,),
                device_id_type=pl.DeviceIdType.MESH,
            )
            rdma.start()
            rdma.wait()
            acc = acc + comm_ref[h + 1, :, :].astype(jnp.float32)

        out_ref[...] = acc

    return pl.pallas_call(
        body,
        out_shape=jax.ShapeDtypeStruct((m, d), jnp.float32),
        in_specs=[pl.BlockSpec(memory_space=pltpu.VMEM)] * 4,
        out_specs=pl.BlockSpec(memory_space=pltpu.VMEM),
        scratch_shapes=[
            pltpu.VMEM((N_DEV, m, d), jnp.bfloat16),
            pltpu.SemaphoreType.DMA((N_DEV - 1,)),
            pltpu.SemaphoreType.DMA((N_DEV - 1,)),
        ],
        compiler_params=pltpu.CompilerParams(collective_id=0),
    )(x, Wg, Wu, Wd)


# device time: 19340 ns/iter; 1.6283x vs baseline; 1.6283x over previous
import jax
import jax.numpy as jnp
from jax import lax
from jax.experimental import pallas as pl
from jax.experimental.pallas import tpu as pltpu

N_DEV = 8
DISTS = (1, 2, 4)


def kernel(x, Wg, Wu, Wd):
    m, k = x.shape
    d = Wd.shape[1]

    def body(x_ref, wg_ref, wu_ref, wd_ref, out_ref,
             send_buf, recv_buf, send_sems, recv_sems):
        my = lax.axis_index("i")

        barrier_sem = pltpu.get_barrier_semaphore()
        for dist in DISTS:
            pl.semaphore_signal(
                barrier_sem, inc=1,
                device_id=(jnp.bitwise_xor(my, dist),),
                device_id_type=pl.DeviceIdType.MESH,
            )
        pl.semaphore_wait(barrier_sem, len(DISTS))

        xb = x_ref[...].astype(jnp.bfloat16)
        gate = jnp.dot(xb, wg_ref[...].astype(jnp.bfloat16),
                       preferred_element_type=jnp.float32)
        up = jnp.dot(xb, wu_ref[...].astype(jnp.bfloat16),
                     preferred_element_type=jnp.float32)
        hidden = (gate * (up * jax.nn.sigmoid(up))).astype(jnp.bfloat16)
        acc = jnp.dot(hidden, wd_ref[...].astype(jnp.bfloat16),
                      preferred_element_type=jnp.float32)

        for r, dist in enumerate(DISTS):
            peer = jnp.bitwise_xor(my, dist)
            send_buf[r, :, :] = acc.astype(jnp.bfloat16)
            rdma = pltpu.make_async_remote_copy(
                src_ref=send_buf.at[r],
                dst_ref=recv_buf.at[r],
                send_sem=send_sems.at[r],
                recv_sem=recv_sems.at[r],
                device_id=(peer,),
                device_id_type=pl.DeviceIdType.MESH,
            )
            rdma.start()
            rdma.wait()
            acc = acc + recv_buf[r, :, :].astype(jnp.float32)

        out_ref[...] = acc

    return pl.pallas_call(
        body,
        out_shape=jax.ShapeDtypeStruct((m, d), jnp.float32),
        in_specs=[pl.BlockSpec(memory_space=pltpu.VMEM)] * 4,
        out_specs=pl.BlockSpec(memory_space=pltpu.VMEM),
        scratch_shapes=[
            pltpu.VMEM((len(DISTS), m, d), jnp.bfloat16),
            pltpu.VMEM((len(DISTS), m, d), jnp.bfloat16),
            pltpu.SemaphoreType.DMA((len(DISTS),)),
            pltpu.SemaphoreType.DMA((len(DISTS),)),
        ],
        compiler_params=pltpu.CompilerParams(collective_id=0),
    )(x, Wg, Wu, Wd)


# device time: 17578 ns/iter; 1.7915x vs baseline; 1.1002x over previous
import jax
import jax.numpy as jnp
from jax import lax
from jax.experimental import pallas as pl
from jax.experimental.pallas import tpu as pltpu

N_DEV = 8
DISTS = (1, 2, 4)
N_HALF = 2


def kernel(x, Wg, Wu, Wd):
    m, k = x.shape
    h = Wg.shape[1]
    d = Wd.shape[1]
    half = d // N_HALF

    def body(x_hbm, wg_hbm, wu_hbm, wd_hbm, out_ref,
             x_v, wg_v, wu_v, wd_v, send_buf, recv_buf,
             in_sems, send_sems, recv_sems):
        my = lax.axis_index("i")

        copies = []
        for i, (src, dst) in enumerate(
                ((x_hbm, x_v), (wg_hbm, wg_v), (wu_hbm, wu_v), (wd_hbm, wd_v))):
            cp = pltpu.make_async_copy(src, dst, in_sems.at[i])
            cp.start()
            copies.append(cp)

        barrier_sem = pltpu.get_barrier_semaphore()
        for dist in DISTS:
            pl.semaphore_signal(
                barrier_sem, inc=1,
                device_id=(jnp.bitwise_xor(my, dist),),
                device_id_type=pl.DeviceIdType.MESH,
            )

        for cp in copies:
            cp.wait()

        xb = x_v[...].astype(jnp.bfloat16)
        gate = jnp.dot(xb, wg_v[...].astype(jnp.bfloat16),
                       preferred_element_type=jnp.float32)
        up = jnp.dot(xb, wu_v[...].astype(jnp.bfloat16),
                     preferred_element_type=jnp.float32)
        hidden = (gate * (up * jax.nn.sigmoid(up))).astype(jnp.bfloat16)
        wd = wd_v[...].astype(jnp.bfloat16)
        acc = [jnp.dot(hidden, wd[:, c * half:(c + 1) * half],
                       preferred_element_type=jnp.float32)
               for c in range(N_HALF)]

        pl.semaphore_wait(barrier_sem, len(DISTS))

        descs = {}

        def start(r, c):
            send_buf[r, c] = acc[c].astype(jnp.bfloat16)
            rd = pltpu.make_async_remote_copy(
                src_ref=send_buf.at[r, c],
                dst_ref=recv_buf.at[r, c],
                send_sem=send_sems.at[r, c],
                recv_sem=recv_sems.at[r, c],
                device_id=(jnp.bitwise_xor(my, DISTS[r]),),
                device_id_type=pl.DeviceIdType.MESH,
            )
            rd.start()
            descs[(r, c)] = rd

        def finish(r, c):
            descs[(r, c)].wait_recv()
            acc[c] = acc[c] + recv_buf[r, c].astype(jnp.float32)

        start(0, 0)
        start(0, 1)
        finish(0, 0); start(1, 0)
        finish(0, 1); start(1, 1)
        finish(1, 0); start(2, 0)
        finish(1, 1); start(2, 1)
        finish(2, 0)
        finish(2, 1)

        for c in range(N_HALF):
            out_ref[:, c * half:(c + 1) * half] = acc[c]

        for key in descs:
            descs[key].wait_send()

    return pl.pallas_call(
        body,
        out_shape=jax.ShapeDtypeStruct((m, d), jnp.float32),
        in_specs=[pl.BlockSpec(memory_space=pl.ANY)] * 4,
        out_specs=pl.BlockSpec(memory_space=pltpu.VMEM),
        scratch_shapes=[
            pltpu.VMEM((m, k), jnp.float32),
            pltpu.VMEM((k, h), jnp.float32),
            pltpu.VMEM((k, h), jnp.float32),
            pltpu.VMEM((h, d), jnp.float32),
            pltpu.VMEM((len(DISTS), N_HALF, m, half), jnp.bfloat16),
            pltpu.VMEM((len(DISTS), N_HALF, m, half), jnp.bfloat16),
            pltpu.SemaphoreType.DMA((4,)),
            pltpu.SemaphoreType.DMA((len(DISTS), N_HALF)),
            pltpu.SemaphoreType.DMA((len(DISTS), N_HALF)),
        ],
        compiler_params=pltpu.CompilerParams(collective_id=0),
    )(x, Wg, Wu, Wd)


# device time: 17080 ns/iter; 1.8437x vs baseline; 1.0292x over previous
import jax
import jax.numpy as jnp
from jax import lax
from jax.experimental import pallas as pl
from jax.experimental.pallas import tpu as pltpu

N_DEV = 8
DISTS = (1, 2, 4)
N_HALF = 2


def kernel(x, Wg, Wu, Wd):
    m, k = x.shape
    h = Wg.shape[1]
    d = Wd.shape[1]
    half = d // N_HALF

    def body(x_hbm, wg_hbm, wu_hbm, wd_hbm, out_hbm,
             x_v, wg_v, wu_v, wd_v, out_v, send_buf, recv_buf,
             in_sems, out_sem, send_sems, recv_sems):
        my = lax.axis_index("i")

        copies = []
        for i, (src, dst) in enumerate(
                ((x_hbm, x_v), (wg_hbm, wg_v), (wu_hbm, wu_v), (wd_hbm, wd_v))):
            cp = pltpu.make_async_copy(src, dst, in_sems.at[i])
            cp.start()
            copies.append(cp)

        barrier_sem = pltpu.get_barrier_semaphore()
        for dist in DISTS:
            pl.semaphore_signal(
                barrier_sem, inc=1,
                device_id=(jnp.bitwise_xor(my, dist),),
                device_id_type=pl.DeviceIdType.MESH,
            )

        for cp in copies:
            cp.wait()

        xb = x_v[...].astype(jnp.bfloat16)
        gate = jnp.dot(xb, wg_v[...].astype(jnp.bfloat16),
                       preferred_element_type=jnp.float32)
        up = jnp.dot(xb, wu_v[...].astype(jnp.bfloat16),
                     preferred_element_type=jnp.float32)
        hidden = (gate * (up * jax.nn.sigmoid(up))).astype(jnp.bfloat16)
        wd = wd_v[...].astype(jnp.bfloat16)

        pl.semaphore_wait(barrier_sem, len(DISTS))

        acc = [None, None]
        descs = {}

        def start(r, c):
            send_buf[r, c] = acc[c].astype(jnp.bfloat16)
            rd = pltpu.make_async_remote_copy(
                src_ref=send_buf.at[r, c],
                dst_ref=recv_buf.at[r, c],
                send_sem=send_sems.at[r, c],
                recv_sem=recv_sems.at[r, c],
                device_id=(jnp.bitwise_xor(my, DISTS[r]),),
                device_id_type=pl.DeviceIdType.MESH,
            )
            rd.start()
            descs[(r, c)] = rd

        def finish(r, c):
            descs[(r, c)].wait_recv()
            acc[c] = acc[c] + recv_buf[r, c].astype(jnp.float32)

        acc[0] = jnp.dot(hidden, wd[:, :half],
                         preferred_element_type=jnp.float32)
        start(0, 0)
        acc[1] = jnp.dot(hidden, wd[:, half:],
                         preferred_element_type=jnp.float32)
        start(0, 1)
        finish(0, 0); start(1, 0)
        finish(0, 1); start(1, 1)
        finish(1, 0); start(2, 0)
        finish(1, 1); start(2, 1)
        finish(2, 0)
        finish(2, 1)

        for c in range(N_HALF):
            out_v[:, c * half:(c + 1) * half] = acc[c]
        out_cp = pltpu.make_async_copy(out_v, out_hbm, out_sem)
        out_cp.start()

        for key in descs:
            descs[key].wait_send()
        out_cp.wait()

    return pl.pallas_call(
        body,
        out_shape=jax.ShapeDtypeStruct((m, d), jnp.float32),
        in_specs=[pl.BlockSpec(memory_space=pltpu.MemorySpace.HBM)] * 4,
        out_specs=pl.BlockSpec(memory_space=pltpu.MemorySpace.HBM),
        scratch_shapes=[
            pltpu.VMEM((m, k), jnp.float32),
            pltpu.VMEM((k, h), jnp.float32),
            pltpu.VMEM((k, h), jnp.float32),
            pltpu.VMEM((h, d), jnp.float32),
            pltpu.VMEM((m, d), jnp.float32),
            pltpu.VMEM((len(DISTS), N_HALF, m, half), jnp.bfloat16),
            pltpu.VMEM((len(DISTS), N_HALF, m, half), jnp.bfloat16),
            pltpu.SemaphoreType.DMA((4,)),
            pltpu.SemaphoreType.DMA,
            pltpu.SemaphoreType.DMA((len(DISTS), N_HALF)),
            pltpu.SemaphoreType.DMA((len(DISTS), N_HALF)),
        ],
        compiler_params=pltpu.CompilerParams(collective_id=0),
    )(x, Wg, Wu, Wd)
